# baseline (speedup 1.0000x reference)
import functools

import jax
import jax.numpy as jnp
from jax import lax
from jax.experimental import pallas as pl
from jax.experimental.pallas import tpu as pltpu
from jax.experimental.pallas import tpu_sc as plsc

D = 64
NC, NS = 2, 16
NW = NC * NS
C = 128
EPS = 1e-12


def _rsqrt(x):
    i = lax.bitcast_convert_type(x, jnp.int32)
    i = jnp.int32(0x5F3759DF) - (i >> 1)
    y = lax.bitcast_convert_type(i, jnp.float32)
    half = x * 0.5
    for _ in range(3):
        y = y * (1.5 - half * y * y)
    return y


def _body(n_tok, wids_hbm, pids_hbm, wtab_hbm, ptab_hbm, lnw_hbm, lnb_hbm,
          out_hbm, widx_v, pidx_v, buf_v, wv, bv, sem):
    wid = lax.axis_index("s") * NC + lax.axis_index("c")
    per_w = n_tok // NW
    n_chunks = per_w // C
    base0 = wid * per_w

    pltpu.sync_copy(lnw_hbm, wv)
    pltpu.sync_copy(lnb_hbm, bv)
    w_vecs = [wv[pl.ds(g * 16, 16)] for g in range(4)]
    b_vecs = [bv[pl.ds(g * 16, 16)] for g in range(4)]

    def chunk_body(ci, carry):
        base = base0 + ci * C
        pltpu.sync_copy(wids_hbm.at[pl.ds(base, C)], widx_v)
        pltpu.sync_copy(pids_hbm.at[pl.ds(base, C)], pidx_v)
        pltpu.async_copy(ptab_hbm.at[pidx_v], buf_v, sem).wait()
        pltpu.async_copy(wtab_hbm.at[widx_v], buf_v, sem, add=True).wait()

        def tok_body(t, carry2):
            e = [buf_v[t, pl.ds(g * 16, 16)] for g in range(4)]
            tot = jnp.sum(e[0] + e[1] + e[2] + e[3])
            u = tot * (1.0 / D)
            d = [eg - u for eg in e]
            s2 = jnp.sum(d[0] * d[0] + d[1] * d[1] + d[2] * d[2] + d[3] * d[3])
            rstd = _rsqrt(s2 * (1.0 / D) + EPS)
            for g in range(4):
                buf_v[t, pl.ds(g * 16, 16)] = d[g] * rstd * w_vecs[g] + b_vecs[g]
            return carry2

        lax.fori_loop(0, C, tok_body, 0)
        pltpu.sync_copy(buf_v, out_hbm.at[pl.ds(base, C)])
        return carry

    lax.fori_loop(0, n_chunks, chunk_body, 0)


def kernel(input_ids, position_ids, word_table, pos_table, ln_weight, ln_bias):
    shape = input_ids.shape
    n_tok = 1
    for s in shape:
        n_tok *= s
    wids = input_ids.reshape((n_tok,)).astype(jnp.int32)
    pids = position_ids.reshape((n_tok,)).astype(jnp.int32)

    mesh = plsc.VectorSubcoreMesh(core_axis_name="c", subcore_axis_name="s",
                                  num_cores=NC, num_subcores=NS)
    run = pl.kernel(
        functools.partial(_body, n_tok),
        out_type=jax.ShapeDtypeStruct((n_tok, D), jnp.float32),
        mesh=mesh,
        compiler_params=pltpu.CompilerParams(needs_layout_passes=False,
                                             use_tc_tiling_on_sc=False),
        scratch_types=[
            pltpu.VMEM((C,), jnp.int32),
            pltpu.VMEM((C,), jnp.int32),
            pltpu.VMEM((C, D), jnp.float32),
            pltpu.VMEM((D,), jnp.float32),
            pltpu.VMEM((D,), jnp.float32),
            pltpu.SemaphoreType.DMA,
        ],
    )
    out = run(wids, pids, word_table, pos_table, ln_weight, ln_bias)
    return out.reshape(shape + (D,))


# speedup vs baseline: 1.6753x; 1.6753x over previous
import functools

import jax
import jax.numpy as jnp
from jax import lax
from jax.experimental import pallas as pl
from jax.experimental.pallas import tpu as pltpu
from jax.experimental.pallas import tpu_sc as plsc

D = 64
NC, NS = 2, 16
NW = NC * NS
CG = 128
C = 256
NBUF = 2
EPS = 1e-12


def _rsqrt(x):
    i = lax.bitcast_convert_type(x, jnp.int32)
    i = jnp.int32(0x5F3759DF) - (i >> 1)
    y = lax.bitcast_convert_type(i, jnp.float32)
    half = x * 0.5
    for _ in range(3):
        y = y * (1.5 - half * y * y)
    return y


def _body(n_tok, wids_hbm, pids_hbm, wtab_hbm, ptab_hbm, lnw_hbm, lnb_hbm,
          out_hbm, widx_v, pidx_v, wbuf0, wbuf1, pbuf0, pbuf1, obuf0, obuf1,
          wv, bv, gsem0, gsem1, ssem0, ssem1):
    wbufs = (wbuf0, wbuf1)
    pbufs = (pbuf0, pbuf1)
    obufs = (obuf0, obuf1)
    gsems = (gsem0, gsem1)
    ssems = (ssem0, ssem1)

    wid = lax.axis_index("s") * NC + lax.axis_index("c")
    per_w = n_tok // NW
    n_chunks = per_w // C
    base0 = wid * per_w

    pltpu.sync_copy(lnw_hbm, wv)
    pltpu.sync_copy(lnb_hbm, bv)
    pltpu.sync_copy(wids_hbm.at[pl.ds(base0, per_w)], widx_v)
    pltpu.sync_copy(pids_hbm.at[pl.ds(base0, per_w)], pidx_v)
    w_vecs = [wv[pl.ds(g * 16, 16)] for g in range(4)]
    b_vecs = [bv[pl.ds(g * 16, 16)] for g in range(4)]

    def gather_descs(ci, b):
        descs = []
        for h in range(C // CG):
            idx_w = widx_v.at[pl.ds(ci * C + h * CG, CG)]
            idx_p = pidx_v.at[pl.ds(ci * C + h * CG, CG)]
            descs.append(pltpu.make_async_copy(
                wtab_hbm.at[idx_w], wbufs[b].at[pl.ds(h * CG, CG)], gsems[b]))
            descs.append(pltpu.make_async_copy(
                ptab_hbm.at[idx_p], pbufs[b].at[pl.ds(h * CG, CG)], gsems[b]))
        return descs

    def issue_gathers(ci, b):
        for desc in gather_descs(ci, b):
            desc.start()

    def wait_gathers(ci, b):
        for desc in gather_descs(ci, b):
            desc.wait()

    def store_desc(ci, b):
        return pltpu.make_async_copy(
            obufs[b], out_hbm.at[pl.ds(base0 + ci * C, C)], ssems[b])

    for b in range(NBUF):
        issue_gathers(b, b)

    def compute_chunk(b):
        U = 4

        def tok_body(ti, carry):
            t0 = ti * U
            for k in range(U):
                t = t0 + k
                e = [wbufs[b][t, pl.ds(g * 16, 16)] + pbufs[b][t, pl.ds(g * 16, 16)]
                     for g in range(4)]
                tot = jnp.sum(e[0] + e[1] + e[2] + e[3])
                u = tot * (1.0 / D)
                d = [eg - u for eg in e]
                s2 = jnp.sum(d[0] * d[0] + d[1] * d[1] + d[2] * d[2] + d[3] * d[3])
                rstd = _rsqrt(s2 * (1.0 / D) + EPS)
                for g in range(4):
                    obufs[b][t, pl.ds(g * 16, 16)] = d[g] * rstd * w_vecs[g] + b_vecs[g]
            return carry
        lax.fori_loop(0, C // U, tok_body, 0)

    def pair_body(g, carry):
        for b in range(NBUF):
            ci = g * NBUF + b
            wait_gathers(ci, b)

            @pl.when(g > 0)
            def _():
                store_desc(ci - NBUF, b).wait()

            compute_chunk(b)
            store_desc(ci, b).start()

            @pl.when(ci + NBUF < n_chunks)
            def _():
                issue_gathers(ci + NBUF, b)
        return carry

    lax.fori_loop(0, n_chunks // NBUF, pair_body, 0)
    for b in range(NBUF):
        store_desc(n_chunks - NBUF + b, b).wait()


def kernel(input_ids, position_ids, word_table, pos_table, ln_weight, ln_bias):
    shape = input_ids.shape
    n_tok = 1
    for s in shape:
        n_tok *= s
    per_w = n_tok // NW
    wids = input_ids.reshape((n_tok,)).astype(jnp.int32)
    pids = position_ids.reshape((n_tok,)).astype(jnp.int32)

    mesh = plsc.VectorSubcoreMesh(core_axis_name="c", subcore_axis_name="s",
                                  num_cores=NC, num_subcores=NS)
    run = pl.kernel(
        functools.partial(_body, n_tok),
        out_type=jax.ShapeDtypeStruct((n_tok, D), jnp.float32),
        mesh=mesh,
        compiler_params=pltpu.CompilerParams(needs_layout_passes=False,
                                             use_tc_tiling_on_sc=False),
        scratch_types=[
            pltpu.VMEM((per_w,), jnp.int32),
            pltpu.VMEM((per_w,), jnp.int32),
            pltpu.VMEM((C, D), jnp.float32),
            pltpu.VMEM((C, D), jnp.float32),
            pltpu.VMEM((C, D), jnp.float32),
            pltpu.VMEM((C, D), jnp.float32),
            pltpu.VMEM((C, D), jnp.float32),
            pltpu.VMEM((C, D), jnp.float32),
            pltpu.VMEM((D,), jnp.float32),
            pltpu.VMEM((D,), jnp.float32),
            pltpu.SemaphoreType.DMA,
            pltpu.SemaphoreType.DMA,
            pltpu.SemaphoreType.DMA,
            pltpu.SemaphoreType.DMA,
        ],
    )
    out = run(wids, pids, word_table, pos_table, ln_weight, ln_bias)
    return out.reshape(shape + (D,))
